# Initial kernel scaffold; baseline (speedup 1.0000x reference)
import jax
import jax.numpy as jnp
from jax.experimental import pallas as pl


def kernel(context_words, emb_table, linear_w, linear_b):
    raise NotImplementedError("write your pallas kernel here")


# speedup vs baseline: 1.0338x; 1.0338x over previous
import functools

import jax
import jax.numpy as jnp
from jax.experimental import pallas as pl
from jax.experimental.pallas import tpu as pltpu


def _cbow_body(ids_ref, emb_ref, w_ref, b_ref, out_ref,
               wbf_ref, gt_ref, ctx_ref, *, C, TB, TV, P, S):
    i = pl.program_id(1)
    j = pl.program_id(2)

    @pl.when(jnp.logical_and(i == 0, j == 0))
    def _cast_w():
        wbf_ref[...] = w_ref[...].astype(jnp.bfloat16)

    @pl.when(j == 0)
    def _gather():
        def group8(g, carry):
            for r in range(8):
                row = g * 8 + r
                base = row * C
                idx = pl.multiple_of(ids_ref[base], P)
                acc = emb_ref[pl.ds(idx, P), :]
                for c in range(1, C):
                    idx = pl.multiple_of(ids_ref[base + c], P)
                    acc = acc + emb_ref[pl.ds(idx, P), :]
                gt_ref[pl.Slice(row, P, S), :] = acc
            return carry

        jax.lax.fori_loop(0, TB // 8, group8, 0, unroll=False)
        for k in range(P):
            ctx_ref[:, k * 128:(k + 1) * 128] = (
                gt_ref[pl.ds(k * S, TB), :].astype(jnp.bfloat16))

    wt = wbf_ref[pl.ds(pl.multiple_of(j * TV, 8), TV), :]
    logits = jax.lax.dot_general(
        ctx_ref[...], wt,
        dimension_numbers=(((1,), (1,)), ((), ())),
        preferred_element_type=jnp.float32)
    out_ref[...] = logits + b_ref[...]


def kernel(context_words, emb_table, linear_w, linear_b):
    B, C = context_words.shape
    V, D = emb_table.shape
    assert linear_w.shape == (V, D) and linear_b.shape == (V,)
    assert V % 128 == 0 and D % 128 == 0

    P = D // 128
    NC = 2
    TB = min(256, B // NC)
    TV = min(2048, V)
    assert B % (TB * NC) == 0 and V % TV == 0 and TB % 8 == 0
    S = TB + 1

    ids = jnp.clip(context_words.reshape(-1).astype(jnp.int32), 0, V - 1) * P
    emb4 = emb_table.astype(jnp.float32).reshape(V * P, 128)
    w = linear_w.astype(jnp.float32)
    b2d = linear_b.reshape(1, V).astype(jnp.float32)

    nb = B // (TB * NC)
    grid = (NC, nb, V // TV)

    body = functools.partial(_cbow_body, C=C, TB=TB, TV=TV, P=P, S=S)
    return pl.pallas_call(
        body,
        out_shape=jax.ShapeDtypeStruct((B, V), jnp.float32),
        grid=grid,
        in_specs=[
            pl.BlockSpec((TB * C,), lambda c, i, j, nb=nb: (c * nb + i,),
                         memory_space=pltpu.MemorySpace.SMEM),
            pl.BlockSpec((V * P, 128), lambda c, i, j: (0, 0),
                         pipeline_mode=pl.Buffered(1)),
            pl.BlockSpec((V, D), lambda c, i, j: (0, 0),
                         pipeline_mode=pl.Buffered(1)),
            pl.BlockSpec((1, TV), lambda c, i, j: (0, j)),
        ],
        out_specs=pl.BlockSpec((TB, TV), lambda c, i, j, nb=nb: (c * nb + i, j)),
        scratch_shapes=[
            pltpu.VMEM((V, D), jnp.bfloat16),
            pltpu.VMEM((S * P, 128), jnp.float32),
            pltpu.VMEM((TB, D), jnp.bfloat16),
        ],
        compiler_params=pltpu.CompilerParams(
            dimension_semantics=("parallel", "arbitrary", "arbitrary"),
            vmem_limit_bytes=60 << 20),
    )(ids, emb4, w, b2d)
